# baseline (speedup 1.0000x reference)
import functools

import jax
import jax.numpy as jnp
from jax import lax
from jax.experimental import pallas as pl
from jax.experimental.pallas import tpu as pltpu
from jax.experimental.pallas import tpu_sc as plsc

N_LATENT = 32
BATCH = 16384
LANES = 16
NUM_CORES = 2
NUM_SUBCORES = 16
NW = NUM_CORES * NUM_SUBCORES
B_PER_W = BATCH // NW
IDX_CHUNK = 128
N_CHUNKS = B_PER_W // IDX_CHUNK


def _mf_kernel(u_hbm, i_hbm, up_hbm, iq_hbm, ub_hbm, ib_hbm, mu_hbm,
               out_hbm,
               u_idx, i_idx, p_v, q_v, ub_v, ib_v, mu_v, out_v, sem):
    wid = lax.axis_index("s") * NUM_CORES + lax.axis_index("c")
    base = wid * B_PER_W

    for j in range(N_CHUNKS):
        pltpu.sync_copy(u_hbm.at[pl.ds(base + j * IDX_CHUNK, IDX_CHUNK)],
                        u_idx.at[j])
        pltpu.sync_copy(i_hbm.at[pl.ds(base + j * IDX_CHUNK, IDX_CHUNK)],
                        i_idx.at[j])
    pltpu.sync_copy(mu_hbm, mu_v)

    copies = []
    for j in range(N_CHUNKS):
        sl = pl.ds(j * IDX_CHUNK, IDX_CHUNK)
        copies.append(pltpu.make_async_copy(up_hbm.at[u_idx.at[j]],
                                            p_v.at[sl], sem))
        copies.append(pltpu.make_async_copy(iq_hbm.at[i_idx.at[j]],
                                            q_v.at[sl], sem))
        copies.append(pltpu.make_async_copy(ub_hbm.at[u_idx.at[j]],
                                            ub_v.at[sl], sem))
        copies.append(pltpu.make_async_copy(ib_hbm.at[i_idx.at[j]],
                                            ib_v.at[sl], sem))
    for c in copies:
        c.start()
    for c in copies:
        c.wait()

    mu_vec = mu_v[...]
    lane = lax.iota(jnp.int32, LANES)

    def body(g, carry):
        sl16 = pl.ds(g * LANES, LANES)
        out_acc = mu_vec + ub_v[sl16] + ib_v[sl16]
        for k in range(LANES):
            b = g * LANES + k
            p0 = p_v[b, pl.ds(0, LANES)]
            p1 = p_v[b, pl.ds(LANES, LANES)]
            q0 = q_v[b, pl.ds(0, LANES)]
            q1 = q_v[b, pl.ds(LANES, LANES)]
            s = jnp.sum(p0 * q0 + p1 * q1)
            out_acc = jnp.where(lane == k, out_acc + s, out_acc)
        out_v[sl16] = out_acc
        return carry

    lax.fori_loop(0, B_PER_W // LANES, body, 0)

    pltpu.sync_copy(out_v, out_hbm.at[pl.ds(base, B_PER_W)])


@jax.jit
def kernel(u, i, user_p, item_q, user_b, item_b, mu):
    mesh = plsc.VectorSubcoreMesh(core_axis_name="c", subcore_axis_name="s")
    run = functools.partial(
        pl.kernel,
        out_type=jax.ShapeDtypeStruct((BATCH,), jnp.float32),
        mesh=mesh,
        scratch_types=[
            pltpu.VMEM((N_CHUNKS, IDX_CHUNK), jnp.int32),
            pltpu.VMEM((N_CHUNKS, IDX_CHUNK), jnp.int32),
            pltpu.VMEM((B_PER_W, N_LATENT), jnp.float32),
            pltpu.VMEM((B_PER_W, N_LATENT), jnp.float32),
            pltpu.VMEM((B_PER_W,), jnp.float32),
            pltpu.VMEM((B_PER_W,), jnp.float32),
            pltpu.VMEM((LANES,), jnp.float32),
            pltpu.VMEM((B_PER_W,), jnp.float32),
            pltpu.SemaphoreType.DMA,
        ],
        compiler_params=pltpu.CompilerParams(needs_layout_passes=False, use_tc_tiling_on_sc=False),
    )(_mf_kernel)
    mu16 = jnp.broadcast_to(mu.astype(jnp.float32), (LANES,))
    return run(u.astype(jnp.int32), i.astype(jnp.int32),
               user_p, item_q,
               user_b.reshape(-1), item_b.reshape(-1), mu16)
